# baseline (speedup 1.0000x reference)
import functools

import jax
import jax.numpy as jnp
from jax import lax
from jax.experimental import pallas as pl
from jax.experimental.pallas import tpu as pltpu
from jax.experimental.pallas import tpu_sc as plsc

_B, _L, _V, _D = 1024, 50, 1000, 64
_N = _B * _L
_LANES = 128
_SUB = _B // _LANES
_NCHUNK = _L * _SUB
_NW = 32
_NSPLIT = 5
_LQ = _L // _NSPLIT
_NCHQ = _LQ * _SUB
_CPW = 3


def _argmax_body(x_ref, o_ref):
    for k in range(_SUB):
        x = x_ref[0, :, k * _LANES:(k + 1) * _LANES]
        row = lax.broadcasted_iota(jnp.int32, x.shape, 0)
        mx = jnp.max(x, axis=0, keepdims=True)
        cand = jnp.where(x == mx, row, _V)
        o_ref[0, k, :] = jnp.min(cand, axis=0)


def _argmax_tc(iw_t):
    return pl.pallas_call(
        _argmax_body,
        grid=(_LQ,),
        in_specs=[pl.BlockSpec((1, _V, _B), lambda i: (i, 0, 0))],
        out_specs=pl.BlockSpec((1, _SUB, _LANES), lambda i: (i, 0, 0)),
        out_shape=jax.ShapeDtypeStruct((_LQ, _SUB, _LANES), jnp.int32),
    )(iw_t)


def _sc_gather(table64, idx2):
    mesh = plsc.VectorSubcoreMesh(core_axis_name="c", subcore_axis_name="s")

    @functools.partial(
        pl.kernel,
        out_type=jax.ShapeDtypeStruct((_LQ * _B, _LANES), jnp.float32),
        mesh=mesh,
        scratch_types=[
            pltpu.VMEM((_CPW, _LANES), jnp.int32),
            pltpu.VMEM((_CPW * _LANES, _D), jnp.float32),
            pltpu.SemaphoreType.DMA,
        ],
        compiler_params=pltpu.CompilerParams(use_tc_tiling_on_sc=False),
    )
    def k(table_hbm, idx_hbm, out_hbm, idx_v, dest, sem_g):
        wid = lax.axis_index("s") * 2 + lax.axis_index("c")
        cbase = (5 * wid) // 2
        pltpu.sync_copy(idx_hbm.at[pl.ds(cbase, _CPW)], idx_v)
        g_cps = [
            pltpu.async_copy(
                table_hbm.at[idx_v.at[j]],
                dest.at[pl.ds(j * _LANES, _LANES)],
                sem_g,
            )
            for j in range(_CPW)
        ]
        for c in g_cps:
            c.wait()
        pltpu.sync_copy(
            dest,
            out_hbm.at[pl.ds(cbase * _LANES, _CPW * _LANES), pl.ds(0, _D)],
        )

    return k(table64, idx2)


def _xpose_body(g_ref, p_ref, o_ref):
    x = g_ref[0][:, 0:_D]
    d_iota = lax.broadcasted_iota(jnp.int32, (_D, _D), 0)
    k_iota = lax.broadcasted_iota(jnp.int32, (_D, _D), 1)
    eye = jnp.where(d_iota == k_iota, 1.0, 0.0).astype(jnp.float32)
    xt = lax.dot_general(eye, x, (((1,), (1,)), ((), ())),
                         precision=lax.Precision.HIGHEST,
                         preferred_element_type=jnp.float32)
    o_ref[0] = xt + jnp.transpose(p_ref[0], (1, 0))


def _xpose_add_tc(g3, pos):
    return pl.pallas_call(
        _xpose_body,
        grid=(_L,),
        in_specs=[
            pl.BlockSpec((1, _B, _LANES), lambda i: (i, 0, 0)),
            pl.BlockSpec((1, 1, _D), lambda i: (i, 0, 0)),
        ],
        out_specs=pl.BlockSpec((1, _D, _B), lambda i: (i, 0, 0)),
        out_shape=jax.ShapeDtypeStruct((_L, _D, _B), jnp.float32),
    )(g3, pos.reshape(_L, 1, _D))


def kernel(index_weights, start_pos, emb_table, pos_emb):
    iw_t = jnp.transpose(index_weights, (1, 2, 0))
    parts = []
    for q in range(_NSPLIT):
        idx = _argmax_tc(iw_t[q * _LQ:(q + 1) * _LQ])
        g3 = _sc_gather(emb_table, idx.reshape(_NCHQ, _LANES))
        parts.append(jnp.transpose(
            g3.reshape(_LQ, _B, _LANES)[:, :, :_D], (1, 0, 2)))
    pos = lax.dynamic_slice_in_dim(pos_emb, start_pos, _L, axis=0)
    return jnp.concatenate(parts, axis=1) + pos[None, :, :]


# speedup vs baseline: 2.0449x; 2.0449x over previous
import functools

import jax
import jax.numpy as jnp
from jax import lax
from jax.experimental import pallas as pl
from jax.experimental.pallas import tpu as pltpu
from jax.experimental.pallas import tpu_sc as plsc

_B, _L, _V, _D = 1024, 50, 1000, 64
_N = _B * _L
_LANES = 128
_SUB = _B // _LANES
_NCHUNK = _L * _SUB
_NW = 32
_CPW = 13


def _argmax_body(x_ref, o_ref):
    for k in range(_SUB):
        x = x_ref[0, :, k * _LANES:(k + 1) * _LANES]
        row = lax.broadcasted_iota(jnp.int32, x.shape, 0)
        mx = jnp.max(x, axis=0, keepdims=True)
        cand = jnp.where(x == mx, row, _V)
        o_ref[0, k, :] = jnp.min(cand, axis=0)


def _argmax_tc(iw_t):
    return pl.pallas_call(
        _argmax_body,
        grid=(_L,),
        in_specs=[pl.BlockSpec((1, _V, _B), lambda i: (i, 0, 0))],
        out_specs=pl.BlockSpec((1, _SUB, _LANES), lambda i: (i, 0, 0)),
        out_shape=jax.ShapeDtypeStruct((_L, _SUB, _LANES), jnp.int32),
    )(iw_t)


def _sc_gather(table64, idx2, posb):
    mesh = plsc.VectorSubcoreMesh(core_axis_name="c", subcore_axis_name="s")

    @functools.partial(
        pl.kernel,
        out_type=jax.ShapeDtypeStruct((_N, _LANES), jnp.float32),
        mesh=mesh,
        scratch_types=[
            pltpu.VMEM((_CPW, _LANES), jnp.int32),
            pltpu.VMEM((_CPW * _LANES, _D), jnp.float32),
            pltpu.SemaphoreType.DMA,
            pltpu.SemaphoreType.DMA,
        ],
        compiler_params=pltpu.CompilerParams(use_tc_tiling_on_sc=False),
    )
    def k(table_hbm, idx_hbm, posb_hbm, out_hbm, idx_v, dest, sem_p, sem_g):
        wid = lax.axis_index("s") * 2 + lax.axis_index("c")
        cbase = (25 * wid) // 2
        pltpu.sync_copy(idx_hbm.at[pl.ds(cbase, _CPW)], idx_v)
        p_cps = [
            pltpu.async_copy(
                posb_hbm.at[pl.ds(((cbase + j) // _SUB) * _LANES, _LANES)],
                dest.at[pl.ds(j * _LANES, _LANES)],
                sem_p,
            )
            for j in range(_CPW)
        ]
        for c in p_cps:
            c.wait()
        g_cps = [
            pltpu.async_copy(
                table_hbm.at[idx_v.at[j]],
                dest.at[pl.ds(j * _LANES, _LANES)],
                sem_g,
                add=True,
            )
            for j in range(_CPW)
        ]
        for c in g_cps:
            c.wait()
        pltpu.sync_copy(
            dest,
            out_hbm.at[pl.ds(cbase * _LANES, _CPW * _LANES), pl.ds(0, _D)],
        )

    return k(table64, idx2, posb)


def _xpose_body(g_ref, p_ref, o_ref):
    x = g_ref[0][:, 0:_D]
    d_iota = lax.broadcasted_iota(jnp.int32, (_D, _D), 0)
    k_iota = lax.broadcasted_iota(jnp.int32, (_D, _D), 1)
    eye = jnp.where(d_iota == k_iota, 1.0, 0.0).astype(jnp.float32)
    xt = lax.dot_general(eye, x, (((1,), (1,)), ((), ())),
                         precision=lax.Precision.HIGHEST,
                         preferred_element_type=jnp.float32)
    o_ref[0] = xt + jnp.transpose(p_ref[0], (1, 0))


def _xpose_add_tc(g3, pos):
    return pl.pallas_call(
        _xpose_body,
        grid=(_L,),
        in_specs=[
            pl.BlockSpec((1, _B, _LANES), lambda i: (i, 0, 0)),
            pl.BlockSpec((1, 1, _D), lambda i: (i, 0, 0)),
        ],
        out_specs=pl.BlockSpec((1, _D, _B), lambda i: (i, 0, 0)),
        out_shape=jax.ShapeDtypeStruct((_L, _D, _B), jnp.float32),
    )(g3, pos.reshape(_L, 1, _D))


def kernel(index_weights, start_pos, emb_table, pos_emb):
    iw_t = jnp.transpose(index_weights, (1, 2, 0))
    idx = _argmax_tc(iw_t)
    pos = lax.dynamic_slice_in_dim(pos_emb, start_pos, _L, axis=0)
    posb = jnp.broadcast_to(pos[:, None, :], (_L, _LANES, _D)).reshape(
        _L * _LANES, _D)
    g3 = _sc_gather(emb_table, idx.reshape(_NCHUNK, _LANES), posb)
    return jnp.transpose(g3.reshape(_L, _B, _LANES)[:, :, :_D], (1, 0, 2))


# speedup vs baseline: 2.2354x; 1.0931x over previous
import functools

import jax
import jax.numpy as jnp
from jax import lax
from jax.experimental import pallas as pl
from jax.experimental.pallas import tpu as pltpu
from jax.experimental.pallas import tpu_sc as plsc

_B, _L, _V, _D = 1024, 50, 1000, 64
_N = _B * _L
_LANES = 128
_SUB = _B // _LANES
_NCHUNK = _L * _SUB
_NW = 32
_CPW = 13


_BL = 2


def _argmax_body(x_ref, o_ref):
    for li in range(_BL):
        for k in range(_SUB):
            x = x_ref[li, :, k * _LANES:(k + 1) * _LANES]
            row = lax.broadcasted_iota(jnp.int32, x.shape, 0)
            mx = jnp.max(x, axis=0, keepdims=True)
            cand = jnp.where(x == mx, row, _V)
            o_ref[li, k, :] = jnp.min(cand, axis=0)


def _argmax_tc(iw_t):
    return pl.pallas_call(
        _argmax_body,
        grid=(_L // _BL,),
        in_specs=[pl.BlockSpec((_BL, _V, _B), lambda i: (i, 0, 0))],
        out_specs=pl.BlockSpec((_BL, _SUB, _LANES), lambda i: (i, 0, 0)),
        out_shape=jax.ShapeDtypeStruct((_L, _SUB, _LANES), jnp.int32),
    )(iw_t)


def _sc_gather(table64, idx2, posb):
    mesh = plsc.VectorSubcoreMesh(core_axis_name="c", subcore_axis_name="s")

    @functools.partial(
        pl.kernel,
        out_type=jax.ShapeDtypeStruct((_N, _LANES), jnp.float32),
        mesh=mesh,
        scratch_types=[
            pltpu.VMEM((_CPW, _LANES), jnp.int32),
            pltpu.VMEM((_CPW * _LANES, _D), jnp.float32),
            pltpu.SemaphoreType.DMA,
            pltpu.SemaphoreType.DMA,
        ],
        compiler_params=pltpu.CompilerParams(use_tc_tiling_on_sc=False),
    )
    def k(table_hbm, idx_hbm, posb_hbm, out_hbm, idx_v, dest, sem_p, sem_g):
        wid = lax.axis_index("s") * 2 + lax.axis_index("c")
        cbase = (25 * wid) // 2
        pltpu.sync_copy(idx_hbm.at[pl.ds(cbase, _CPW)], idx_v)
        p_cps = [
            pltpu.async_copy(
                posb_hbm.at[pl.ds(((cbase + j) // _SUB) * _LANES, _LANES)],
                dest.at[pl.ds(j * _LANES, _LANES)],
                sem_p,
            )
            for j in range(_CPW)
        ]
        for c in p_cps:
            c.wait()
        g_cps = [
            pltpu.async_copy(
                table_hbm.at[idx_v.at[j]],
                dest.at[pl.ds(j * _LANES, _LANES)],
                sem_g,
                add=True,
            )
            for j in range(_CPW)
        ]
        for c in g_cps:
            c.wait()
        pltpu.sync_copy(
            dest,
            out_hbm.at[pl.ds(cbase * _LANES, _CPW * _LANES), pl.ds(0, _D)],
        )

    return k(table64, idx2, posb)


def _xpose_body(g_ref, p_ref, o_ref):
    x = g_ref[0][:, 0:_D]
    d_iota = lax.broadcasted_iota(jnp.int32, (_D, _D), 0)
    k_iota = lax.broadcasted_iota(jnp.int32, (_D, _D), 1)
    eye = jnp.where(d_iota == k_iota, 1.0, 0.0).astype(jnp.float32)
    xt = lax.dot_general(eye, x, (((1,), (1,)), ((), ())),
                         precision=lax.Precision.HIGHEST,
                         preferred_element_type=jnp.float32)
    o_ref[0] = xt + jnp.transpose(p_ref[0], (1, 0))


def _xpose_add_tc(g3, pos):
    return pl.pallas_call(
        _xpose_body,
        grid=(_L,),
        in_specs=[
            pl.BlockSpec((1, _B, _LANES), lambda i: (i, 0, 0)),
            pl.BlockSpec((1, 1, _D), lambda i: (i, 0, 0)),
        ],
        out_specs=pl.BlockSpec((1, _D, _B), lambda i: (i, 0, 0)),
        out_shape=jax.ShapeDtypeStruct((_L, _D, _B), jnp.float32),
    )(g3, pos.reshape(_L, 1, _D))


def kernel(index_weights, start_pos, emb_table, pos_emb):
    iw_t = jnp.transpose(index_weights, (1, 2, 0))
    idx = _argmax_tc(iw_t)
    pos = lax.dynamic_slice_in_dim(pos_emb, start_pos, _L, axis=0)
    posb = jnp.broadcast_to(pos[:, None, :], (_L, _LANES, _D)).reshape(
        _L * _LANES, _D)
    g3 = _sc_gather(emb_table, idx.reshape(_NCHUNK, _LANES), posb)
    return jnp.transpose(g3.reshape(_L, _B, _LANES)[:, :, :_D], (1, 0, 2))


# speedup vs baseline: 2.2934x; 1.0260x over previous
import functools

import jax
import jax.numpy as jnp
from jax import lax
from jax.experimental import pallas as pl
from jax.experimental.pallas import tpu as pltpu
from jax.experimental.pallas import tpu_sc as plsc

_B, _L, _V, _D = 1024, 50, 1000, 64
_N = _B * _L
_LANES = 128
_SUB = _B // _LANES
_NCHUNK = _L * _SUB
_NW = 32
_CPW = 13


_BL = 5


def _argmax_body(x_ref, o_ref):
    for li in range(_BL):
        for k in range(_SUB):
            x = x_ref[li, :, k * _LANES:(k + 1) * _LANES]
            row = lax.broadcasted_iota(jnp.int32, x.shape, 0)
            mx = jnp.max(x, axis=0, keepdims=True)
            cand = jnp.where(x == mx, row, _V)
            o_ref[li, k, :] = jnp.min(cand, axis=0)


def _argmax_tc(iw_t):
    return pl.pallas_call(
        _argmax_body,
        grid=(_L // _BL,),
        in_specs=[pl.BlockSpec((_BL, _V, _B), lambda i: (i, 0, 0))],
        out_specs=pl.BlockSpec((_BL, _SUB, _LANES), lambda i: (i, 0, 0)),
        out_shape=jax.ShapeDtypeStruct((_L, _SUB, _LANES), jnp.int32),
    )(iw_t)


def _sc_gather(table64, idx2, posb):
    mesh = plsc.VectorSubcoreMesh(core_axis_name="c", subcore_axis_name="s")

    @functools.partial(
        pl.kernel,
        out_type=jax.ShapeDtypeStruct((_N, _LANES), jnp.float32),
        mesh=mesh,
        scratch_types=[
            pltpu.VMEM((_CPW, _LANES), jnp.int32),
            pltpu.VMEM((_CPW * _LANES, _D), jnp.float32),
            pltpu.SemaphoreType.DMA,
            pltpu.SemaphoreType.DMA,
        ],
        compiler_params=pltpu.CompilerParams(use_tc_tiling_on_sc=False),
    )
    def k(table_hbm, idx_hbm, posb_hbm, out_hbm, idx_v, dest, sem_p, sem_g):
        wid = lax.axis_index("s") * 2 + lax.axis_index("c")
        cbase = (25 * wid) // 2
        pltpu.sync_copy(idx_hbm.at[pl.ds(cbase, _CPW)], idx_v)
        p_cps = [
            pltpu.async_copy(
                posb_hbm.at[pl.ds(((cbase + j) // _SUB) * _LANES, _LANES)],
                dest.at[pl.ds(j * _LANES, _LANES)],
                sem_p,
            )
            for j in range(_CPW)
        ]
        for c in p_cps:
            c.wait()
        g_cps = [
            pltpu.async_copy(
                table_hbm.at[idx_v.at[j]],
                dest.at[pl.ds(j * _LANES, _LANES)],
                sem_g,
                add=True,
            )
            for j in range(_CPW)
        ]
        for c in g_cps:
            c.wait()
        pltpu.sync_copy(
            dest,
            out_hbm.at[pl.ds(cbase * _LANES, _CPW * _LANES), pl.ds(0, _D)],
        )

    return k(table64, idx2, posb)


def _xpose_body(g_ref, p_ref, o_ref):
    x = g_ref[0][:, 0:_D]
    d_iota = lax.broadcasted_iota(jnp.int32, (_D, _D), 0)
    k_iota = lax.broadcasted_iota(jnp.int32, (_D, _D), 1)
    eye = jnp.where(d_iota == k_iota, 1.0, 0.0).astype(jnp.float32)
    xt = lax.dot_general(eye, x, (((1,), (1,)), ((), ())),
                         precision=lax.Precision.HIGHEST,
                         preferred_element_type=jnp.float32)
    o_ref[0] = xt + jnp.transpose(p_ref[0], (1, 0))


def _xpose_add_tc(g3, pos):
    return pl.pallas_call(
        _xpose_body,
        grid=(_L,),
        in_specs=[
            pl.BlockSpec((1, _B, _LANES), lambda i: (i, 0, 0)),
            pl.BlockSpec((1, 1, _D), lambda i: (i, 0, 0)),
        ],
        out_specs=pl.BlockSpec((1, _D, _B), lambda i: (i, 0, 0)),
        out_shape=jax.ShapeDtypeStruct((_L, _D, _B), jnp.float32),
    )(g3, pos.reshape(_L, 1, _D))


def kernel(index_weights, start_pos, emb_table, pos_emb):
    iw_t = jnp.transpose(index_weights, (1, 2, 0))
    idx = _argmax_tc(iw_t)
    pos = lax.dynamic_slice_in_dim(pos_emb, start_pos, _L, axis=0)
    posb = jnp.broadcast_to(pos[:, None, :], (_L, _LANES, _D)).reshape(
        _L * _LANES, _D)
    g3 = _sc_gather(emb_table, idx.reshape(_NCHUNK, _LANES), posb)
    return jnp.transpose(g3.reshape(_L, _B, _LANES)[:, :, :_D], (1, 0, 2))
